# Initial kernel scaffold; baseline (speedup 1.0000x reference)
import jax
import jax.numpy as jnp
from jax.experimental import pallas as pl


def kernel(node_features, edge_index, num_nodes, W_in, b_in, W0, b0, W1, b1):
    raise NotImplementedError("write your pallas kernel here")


# speedup vs baseline: 2.6114x; 2.6114x over previous
import functools

import jax
import jax.numpy as jnp
from jax import lax
from jax.experimental import pallas as pl
from jax.experimental.pallas import tpu as pltpu
from jax.experimental.pallas import tpu_sc as plsc

N = 10000
F = 256
H = 512
E = 160000
LANE = 128
C = H // LANE

NC = 2
NS = 16
CPC = C // NC
B = 128
NB = 80
EPW = NB * B
EPAD = NS * EPW
ACC_ROWS = 10240
NBH = 40
OUT_STRIPE = 624


BR = 2000
NI = N // BR


def _proj_in_body(x_ref, w_ref, b_ref, out_ref):
    y = jnp.dot(x_ref[...], w_ref[0], preferred_element_type=jnp.float32)
    out_ref[0] = jnp.maximum(y + b_ref[0], 0.0)


def _proj_in(x, w_r, b_r):
    return pl.pallas_call(
        _proj_in_body,
        grid=(C, NI),
        in_specs=[
            pl.BlockSpec((BR, F), lambda c, i: (i, 0)),
            pl.BlockSpec((1, F, LANE), lambda c, i: (c, 0, 0)),
            pl.BlockSpec((1, 1, LANE), lambda c, i: (c, 0, 0)),
        ],
        out_specs=pl.BlockSpec((1, BR, LANE), lambda c, i: (c, i, 0)),
        out_shape=jax.ShapeDtypeStruct((C, N, LANE), jnp.float32),
        compiler_params=pltpu.CompilerParams(
            dimension_semantics=("parallel", "parallel")),
    )(x, w_r, b_r)


def _layer_body(m_ref, w_ref, b_ref, hprev_ref, out_ref, acc_ref):
    k = pl.program_id(2)

    @pl.when(k == 0)
    def _():
        acc_ref[...] = jnp.dot(m_ref[0], w_ref[0, 0],
                               preferred_element_type=jnp.float32)

    @pl.when(k > 0)
    def _():
        acc_ref[...] += jnp.dot(m_ref[0], w_ref[0, 0],
                                preferred_element_type=jnp.float32)

    @pl.when(k == C - 1)
    def _():
        out_ref[0] = hprev_ref[0] + jnp.maximum(acc_ref[...] + b_ref[0], 0.0)


def _layer_mm(m, w_r, b_r, hprev):
    return pl.pallas_call(
        _layer_body,
        grid=(C, NI, C),
        in_specs=[
            pl.BlockSpec((1, BR, LANE), lambda c, i, k: (k, i, 0)),
            pl.BlockSpec((1, 1, LANE, LANE), lambda c, i, k: (k, c, 0, 0)),
            pl.BlockSpec((1, 1, LANE), lambda c, i, k: (c, 0, 0)),
            pl.BlockSpec((1, BR, LANE), lambda c, i, k: (c, i, 0)),
        ],
        out_specs=pl.BlockSpec((1, BR, LANE), lambda c, i, k: (c, i, 0)),
        out_shape=jax.ShapeDtypeStruct((C, N, LANE), jnp.float32),
        scratch_shapes=[pltpu.VMEM((BR, LANE), jnp.float32)],
        compiler_params=pltpu.CompilerParams(
            dimension_semantics=("parallel", "parallel", "arbitrary")),
    )(m, w_r, b_r, hprev)


def _final_body(m_ref, w_ref, b_ref, hprev_ref, out_ref, acc_ref):
    i = pl.program_id(1)
    k = pl.program_id(2)

    @pl.when(k == 0)
    def _():
        acc_ref[...] = jnp.dot(m_ref[0], w_ref[0, 0],
                               preferred_element_type=jnp.float32)

    @pl.when(k > 0)
    def _():
        acc_ref[...] += jnp.dot(m_ref[0], w_ref[0, 0],
                                preferred_element_type=jnp.float32)

    @pl.when(k == C - 1)
    def _():
        h2 = hprev_ref[0] + jnp.maximum(acc_ref[...] + b_ref[0], 0.0)
        part = jnp.sum(h2, axis=0, keepdims=True) * (1.0 / N)
        tile = jnp.broadcast_to(part, (8, LANE))

        @pl.when(i == 0)
        def _():
            out_ref[0] = tile

        @pl.when(i > 0)
        def _():
            out_ref[0] += tile


def _final_mm(m, w_r, b_r, hprev):
    return pl.pallas_call(
        _final_body,
        grid=(C, NI, C),
        in_specs=[
            pl.BlockSpec((1, BR, LANE), lambda c, i, k: (k, i, 0)),
            pl.BlockSpec((1, 1, LANE, LANE), lambda c, i, k: (k, c, 0, 0)),
            pl.BlockSpec((1, 1, LANE), lambda c, i, k: (c, 0, 0)),
            pl.BlockSpec((1, BR, LANE), lambda c, i, k: (c, i, 0)),
        ],
        out_specs=pl.BlockSpec((1, 8, LANE), lambda c, i, k: (c, 0, 0)),
        out_shape=jax.ShapeDtypeStruct((C, 8, LANE), jnp.float32),
        scratch_shapes=[pltpu.VMEM((BR, LANE), jnp.float32)],
        compiler_params=pltpu.CompilerParams(
            dimension_semantics=("parallel", "arbitrary", "arbitrary")),
    )(m, w_r, b_r, hprev)


def _sc_scatter_body(h_flat, srcoff_hbm, dst_hbm, out_hbm,
                     src_v, dst_v, buf_a, buf_b, acc, sem_a, sem_b):
    cid = lax.axis_index("c")
    sid = lax.axis_index("s")

    for kk in range(CPC):
        chunk = cid * CPC + kk

        def _zero(t, carry):
            buf_a[t // 8, pl.ds((t % 8) * 16, 16)] = jnp.zeros((16,), jnp.float32)
            return carry
        lax.fori_loop(0, B * 8, _zero, 0)
        for t in range(5):
            pltpu.sync_copy(buf_a, acc.at[pl.ds(sid * 640 + t * B, B)])

        plsc.subcore_barrier()

        def _fire(j, buf, sem):
            pltpu.async_copy(h_flat.at[src_v.at[j]], buf, sem)

        def _wait(buf, sem):
            pltpu.make_async_copy(h_flat.at[src_v.at[0]], buf, sem).wait()

        for hh in range(2):
            pltpu.sync_copy(srcoff_hbm.at[chunk, sid, pl.ds(hh * NBH, NBH)],
                            src_v)
            pltpu.sync_copy(dst_hbm.at[sid, pl.ds(hh * NBH, NBH)], dst_v)

            _fire(0, buf_a, sem_a)
            _fire(1, buf_b, sem_b)

            def _step(it, carry):
                j = it * 2
                _wait(buf_a, sem_a)
                pltpu.sync_copy(buf_a, acc.at[dst_v.at[j]], add=True)

                @pl.when(j + 2 < NBH)
                def _():
                    _fire(j + 2, buf_a, sem_a)

                _wait(buf_b, sem_b)
                pltpu.sync_copy(buf_b, acc.at[dst_v.at[j + 1]], add=True)

                @pl.when(j + 3 < NBH)
                def _():
                    _fire(j + 3, buf_b, sem_b)
                return carry

            lax.fori_loop(0, NBH // 2, _step, 0)

        plsc.subcore_barrier()

        pltpu.sync_copy(acc.at[pl.ds(sid * OUT_STRIPE, OUT_STRIPE)],
                        out_hbm.at[chunk].at[pl.ds(sid * OUT_STRIPE, OUT_STRIPE)])

        @pl.when(sid == 0)
        def _():
            pltpu.sync_copy(acc.at[pl.ds(NS * OUT_STRIPE, N - NS * OUT_STRIPE)],
                            out_hbm.at[chunk].at[pl.ds(NS * OUT_STRIPE,
                                                       N - NS * OUT_STRIPE)])

        plsc.subcore_barrier()


_sc_scatter = functools.partial(
    pl.kernel,
    out_type=jax.ShapeDtypeStruct((C, N, LANE), jnp.float32),
    mesh=plsc.VectorSubcoreMesh(
        core_axis_name="c", subcore_axis_name="s",
        num_cores=NC, num_subcores=NS),
    scratch_types=[
        pltpu.VMEM((NBH, B), jnp.int32),
        pltpu.VMEM((NBH, B), jnp.int32),
        pltpu.VMEM((B, LANE), jnp.float32),
        pltpu.VMEM((B, LANE), jnp.float32),
        pltpu.VMEM_SHARED((ACC_ROWS, LANE), jnp.float32),
        pltpu.SemaphoreType.DMA,
        pltpu.SemaphoreType.DMA,
    ],
)(_sc_scatter_body)


def _scatter_edges(h_cm, srcoff, dst_r):
    return _sc_scatter(h_cm.reshape(C * N, LANE), srcoff, dst_r)


def kernel(node_features, edge_index, num_nodes, W_in, b_in, W0, b0, W1, b1):
    w_in_r = W_in.reshape(F, C, LANE).transpose(1, 0, 2)
    b_in_r = b_in.reshape(C, 1, LANE)
    w0_r = W0.reshape(C, LANE, C, LANE).transpose(0, 2, 1, 3)
    b0_r = b0.reshape(C, 1, LANE)
    w1_r = W1.reshape(C, LANE, C, LANE).transpose(0, 2, 1, 3)
    b1_r = b1.reshape(C, 1, LANE)

    src = edge_index[0]
    dst = edge_index[1]
    pad = EPAD - E
    src_p = jnp.concatenate([src, jnp.zeros((pad,), jnp.int32)])
    dst_p = jnp.concatenate([dst, jnp.full((pad,), N, jnp.int32)])
    dst_r = dst_p.reshape(NS, NB, B)
    srcoff = (src_p.reshape(1, NS, NB, B)
              + (jnp.arange(C, dtype=jnp.int32) * N).reshape(C, 1, 1, 1))

    h0 = _proj_in(node_features, w_in_r, b_in_r)
    m0 = _scatter_edges(h0, srcoff, dst_r)
    h1 = _layer_mm(m0, w0_r, b0_r, h0)
    m1 = _scatter_edges(h1, srcoff, dst_r)
    sums = _final_mm(m1, w1_r, b1_r, h1)
    out = sums[:, 0, :].reshape(H)
    return jnp.where(num_nodes == 0, jnp.zeros((H,), node_features.dtype), out)
